# baseline (speedup 1.0000x reference)
import functools

import jax
import jax.numpy as jnp
from jax import lax
from jax.experimental import pallas as pl
from jax.experimental.pallas import tpu as pltpu
from jax.experimental.pallas import tpu_sc as plsc

_NC = 2
_NS = 16
_NW = _NC * _NS
_CB = 8


def _make_sc_call(R, n, d):
    rows_pw = R // _NW
    nch = rows_pw // _CB
    mesh = plsc.VectorSubcoreMesh(
        core_axis_name="c", subcore_axis_name="s", num_cores=_NC, num_subcores=_NS
    )

    def body(x_hbm, emb_hbm, us_hbm, bs_hbm, is_hbm, out_hbm,
             xbuf, tab, usv, bsv, isv):
        wid = lax.axis_index("s") * _NC + lax.axis_index("c")
        row0 = wid * rows_pw
        pos0 = lax.rem(row0, n)

        pltpu.sync_copy(emb_hbm, tab)
        pltpu.sync_copy(us_hbm, usv)
        pltpu.sync_copy(bs_hbm, bsv)
        pltpu.sync_copy(is_hbm, isv)
        uv = usv[...]
        bv = bsv[...]
        iv = isv[...]

        def chunk(c, carry):
            start = (row0 + c * _CB) * d
            pltpu.sync_copy(x_hbm.at[pl.ds(start, _CB * d)], xbuf)
            base_pos = pos0 + c * _CB
            masks = []
            for r in range(_CB):
                p16 = jnp.full((16,), base_pos + r, jnp.int32)
                masks.append((p16 < uv, p16 < bv, p16 < iv))
            for k in range(d // 16):
                t0k = tab[pl.ds(k * 16, 16)]
                t1k = tab[pl.ds(d + k * 16, 16)]
                t2k = tab[pl.ds(2 * d + k * 16, 16)]
                for r in range(_CB):
                    c1, c2, c3 = masks[r]
                    e = jnp.where(c1, t0k, jnp.where(c2, t1k, jnp.where(c3, t0k, t2k)))
                    plsc.addupdate(xbuf.at[pl.ds(r * d + k * 16, 16)], e)
            pltpu.sync_copy(xbuf, out_hbm.at[pl.ds(start, _CB * d)])
            return carry

        lax.fori_loop(0, nch, chunk, 0)

    return pl.kernel(
        body,
        out_type=jax.ShapeDtypeStruct((R * d,), jnp.float32),
        mesh=mesh,
        scratch_types=[
            pltpu.VMEM((_CB * d,), jnp.float32),
            pltpu.VMEM((3 * d,), jnp.float32),
            pltpu.VMEM((16,), jnp.int32),
            pltpu.VMEM((16,), jnp.int32),
            pltpu.VMEM((16,), jnp.int32),
        ],
    )


def kernel(x, emb_table, seq_len, front, back, keyframe_gap):
    batch, n, d = x.shape
    R = batch * n
    seq_len = jnp.asarray(seq_len, jnp.int32)
    front = jnp.asarray(front, jnp.int32)
    back = jnp.asarray(back, jnp.int32)
    keyframe_gap = jnp.asarray(keyframe_gap, jnp.int32)
    ignored_len = seq_len - front - back - keyframe_gap
    us16 = jnp.full((16,), front, jnp.int32)
    bs16 = jnp.full((16,), front + keyframe_gap, jnp.int32)
    is16 = jnp.full((16,), seq_len - ignored_len, jnp.int32)

    x_flat = x.reshape(R * d)
    emb_flat = emb_table.reshape(3 * d)
    out = _make_sc_call(R, n, d)(x_flat, emb_flat, us16, bs16, is16)
    return out.reshape(batch, n, d)


# speedup vs baseline: 1.0653x; 1.0653x over previous
import jax
import jax.numpy as jnp
from jax import lax
from jax.experimental import pallas as pl
from jax.experimental.pallas import tpu as pltpu
from jax.experimental.pallas import tpu_sc as plsc

_NC = 2
_NS = 16
_NW = _NC * _NS
_CB = 32
_SB = 8


def _compute_chunk(buf, tab, uv, bv, iv, cpos0, d):
    def sub_block(s, carry):
        base = s * (_SB * d)
        masks = []
        for r in range(_SB):
            p16 = jnp.full((16,), cpos0 + s * _SB + r, jnp.int32)
            masks.append((p16 < uv, p16 < bv, p16 < iv))
        for k in range(d // 16):
            t0k = tab[pl.ds(k * 16, 16)]
            t1k = tab[pl.ds(d + k * 16, 16)]
            t2k = tab[pl.ds(2 * d + k * 16, 16)]
            for r in range(_SB):
                c1, c2, c3 = masks[r]
                e = jnp.where(c1, t0k, jnp.where(c2, t1k, jnp.where(c3, t0k, t2k)))
                plsc.addupdate(buf.at[pl.ds(base + r * d + k * 16, 16)], e)
        return carry

    lax.fori_loop(0, _CB // _SB, sub_block, 0)


def _make_sc_call(R, n, d):
    rows_pw = R // _NW
    nch = rows_pw // _CB
    mesh = plsc.VectorSubcoreMesh(
        core_axis_name="c", subcore_axis_name="s", num_cores=_NC, num_subcores=_NS
    )

    def body(x_hbm, emb_hbm, us_hbm, bs_hbm, is_hbm, out_hbm,
             xbuf0, xbuf1, tab, usv, bsv, isv,
             semi0, semi1, semo0, semo1):
        wid = lax.axis_index("s") * _NC + lax.axis_index("c")
        row0 = wid * rows_pw
        pos0 = lax.rem(row0, n)

        pltpu.sync_copy(emb_hbm, tab)
        pltpu.sync_copy(us_hbm, usv)
        pltpu.sync_copy(bs_hbm, bsv)
        pltpu.sync_copy(is_hbm, isv)
        uv = usv[...]
        bv = bsv[...]
        iv = isv[...]

        bufs = (xbuf0, xbuf1)
        semis = (semi0, semi1)
        semos = (semo0, semo1)

        def start_in(c, buf, sem):
            start = (row0 + c * _CB) * d
            return pltpu.async_copy(x_hbm.at[pl.ds(start, _CB * d)], buf, sem)

        def start_out(c, buf, sem):
            start = (row0 + c * _CB) * d
            return pltpu.async_copy(buf, out_hbm.at[pl.ds(start, _CB * d)], sem)

        def wait_in(c, buf, sem):
            start = (row0 + c * _CB) * d
            pltpu.make_async_copy(x_hbm.at[pl.ds(start, _CB * d)], buf, sem).wait()

        def wait_out(c, buf, sem):
            start = (row0 + c * _CB) * d
            pltpu.make_async_copy(buf, out_hbm.at[pl.ds(start, _CB * d)], sem).wait()

        start_in(0, bufs[0], semis[0])

        def group(g, carry):
            for b in range(2):
                c = 2 * g + b
                wait_in(c, bufs[b], semis[b])
                nxt = 1 - b
                if b == 0:
                    @pl.when(g >= 1)
                    def _():
                        wait_out(c - 1, bufs[nxt], semos[nxt])
                    start_in(c + 1, bufs[nxt], semis[nxt])
                else:
                    wait_out(c - 1, bufs[nxt], semos[nxt])

                    @pl.when(g < nch // 2 - 1)
                    def _():
                        start_in(c + 1, bufs[nxt], semis[nxt])
                _compute_chunk(bufs[b], tab, uv, bv, iv, pos0 + c * _CB, d)
                start_out(c, bufs[b], semos[b])
            return carry

        lax.fori_loop(0, nch // 2, group, 0)
        wait_out(nch - 1, bufs[1], semos[1])

    return pl.kernel(
        body,
        out_type=jax.ShapeDtypeStruct((R * d,), jnp.float32),
        mesh=mesh,
        scratch_types=[
            pltpu.VMEM((_CB * d,), jnp.float32),
            pltpu.VMEM((_CB * d,), jnp.float32),
            pltpu.VMEM((3 * d,), jnp.float32),
            pltpu.VMEM((16,), jnp.int32),
            pltpu.VMEM((16,), jnp.int32),
            pltpu.VMEM((16,), jnp.int32),
            pltpu.SemaphoreType.DMA,
            pltpu.SemaphoreType.DMA,
            pltpu.SemaphoreType.DMA,
            pltpu.SemaphoreType.DMA,
        ],
    )


def kernel(x, emb_table, seq_len, front, back, keyframe_gap):
    batch, n, d = x.shape
    R = batch * n
    seq_len = jnp.asarray(seq_len, jnp.int32)
    front = jnp.asarray(front, jnp.int32)
    back = jnp.asarray(back, jnp.int32)
    keyframe_gap = jnp.asarray(keyframe_gap, jnp.int32)
    ignored_len = seq_len - front - back - keyframe_gap
    us16 = jnp.full((16,), front, jnp.int32)
    bs16 = jnp.full((16,), front + keyframe_gap, jnp.int32)
    is16 = jnp.full((16,), seq_len - ignored_len, jnp.int32)

    x_flat = x.reshape(R * d)
    emb_flat = emb_table.reshape(3 * d)
    out = _make_sc_call(R, n, d)(x_flat, emb_flat, us16, bs16, is16)
    return out.reshape(batch, n, d)


# speedup vs baseline: 1.3101x; 1.2297x over previous
import jax
import jax.numpy as jnp
from jax import lax
from jax.experimental import pallas as pl
from jax.experimental.pallas import tpu as pltpu
from jax.experimental.pallas import tpu_sc as plsc

_NC = 2
_NS = 16
_NW = _NC * _NS
_CB = 32
_SB = 8


def _compute_chunk(buf, tab, uv, bv, iv, cpos0, d):
    def sub_block(s, carry):
        base = s * (_SB * d)
        masks = []
        for r in range(_SB):
            p16 = jnp.full((16,), cpos0 + s * _SB + r, jnp.int32)
            masks.append((p16 < uv, p16 < bv, p16 < iv))

        @plsc.parallel_loop(0, d // 16, unroll=2)
        def _cols(k):
            off = k * 16
            t0k = tab[pl.ds(off, 16)]
            t1k = tab[pl.ds(d + off, 16)]
            t2k = tab[pl.ds(2 * d + off, 16)]
            for r in range(_SB):
                c1, c2, c3 = masks[r]
                e = jnp.where(c1, t0k, jnp.where(c2, t1k, jnp.where(c3, t0k, t2k)))
                plsc.addupdate(buf.at[pl.ds(base + r * d + off, 16)], e)

        return carry

    lax.fori_loop(0, _CB // _SB, sub_block, 0)


def _make_sc_call(R, n, d):
    rows_pw = R // _NW
    nch = rows_pw // _CB
    mesh = plsc.VectorSubcoreMesh(
        core_axis_name="c", subcore_axis_name="s", num_cores=_NC, num_subcores=_NS
    )

    def body(x_hbm, emb_hbm, us_hbm, bs_hbm, is_hbm, out_hbm,
             xbuf0, xbuf1, tab, usv, bsv, isv,
             semi0, semi1, semo0, semo1):
        wid = lax.axis_index("s") * _NC + lax.axis_index("c")
        row0 = wid * rows_pw
        pos0 = lax.rem(row0, n)

        pltpu.sync_copy(emb_hbm, tab)
        pltpu.sync_copy(us_hbm, usv)
        pltpu.sync_copy(bs_hbm, bsv)
        pltpu.sync_copy(is_hbm, isv)
        uv = usv[...]
        bv = bsv[...]
        iv = isv[...]

        bufs = (xbuf0, xbuf1)
        semis = (semi0, semi1)
        semos = (semo0, semo1)

        def start_in(c, buf, sem):
            start = (row0 + c * _CB) * d
            return pltpu.async_copy(x_hbm.at[pl.ds(start, _CB * d)], buf, sem)

        def start_out(c, buf, sem):
            start = (row0 + c * _CB) * d
            return pltpu.async_copy(buf, out_hbm.at[pl.ds(start, _CB * d)], sem)

        def wait_in(c, buf, sem):
            start = (row0 + c * _CB) * d
            pltpu.make_async_copy(x_hbm.at[pl.ds(start, _CB * d)], buf, sem).wait()

        def wait_out(c, buf, sem):
            start = (row0 + c * _CB) * d
            pltpu.make_async_copy(buf, out_hbm.at[pl.ds(start, _CB * d)], sem).wait()

        start_in(0, bufs[0], semis[0])

        def group(g, carry):
            for b in range(2):
                c = 2 * g + b
                wait_in(c, bufs[b], semis[b])
                nxt = 1 - b
                if b == 0:
                    @pl.when(g >= 1)
                    def _():
                        wait_out(c - 1, bufs[nxt], semos[nxt])
                    start_in(c + 1, bufs[nxt], semis[nxt])
                else:
                    wait_out(c - 1, bufs[nxt], semos[nxt])

                    @pl.when(g < nch // 2 - 1)
                    def _():
                        start_in(c + 1, bufs[nxt], semis[nxt])
                _compute_chunk(bufs[b], tab, uv, bv, iv, pos0 + c * _CB, d)
                start_out(c, bufs[b], semos[b])
            return carry

        lax.fori_loop(0, nch // 2, group, 0)
        wait_out(nch - 1, bufs[1], semos[1])

    return pl.kernel(
        body,
        out_type=jax.ShapeDtypeStruct((R * d,), jnp.float32),
        mesh=mesh,
        scratch_types=[
            pltpu.VMEM((_CB * d,), jnp.float32),
            pltpu.VMEM((_CB * d,), jnp.float32),
            pltpu.VMEM((3 * d,), jnp.float32),
            pltpu.VMEM((16,), jnp.int32),
            pltpu.VMEM((16,), jnp.int32),
            pltpu.VMEM((16,), jnp.int32),
            pltpu.SemaphoreType.DMA,
            pltpu.SemaphoreType.DMA,
            pltpu.SemaphoreType.DMA,
            pltpu.SemaphoreType.DMA,
        ],
    )


def kernel(x, emb_table, seq_len, front, back, keyframe_gap):
    batch, n, d = x.shape
    R = batch * n
    seq_len = jnp.asarray(seq_len, jnp.int32)
    front = jnp.asarray(front, jnp.int32)
    back = jnp.asarray(back, jnp.int32)
    keyframe_gap = jnp.asarray(keyframe_gap, jnp.int32)
    ignored_len = seq_len - front - back - keyframe_gap
    us16 = jnp.full((16,), front, jnp.int32)
    bs16 = jnp.full((16,), front + keyframe_gap, jnp.int32)
    is16 = jnp.full((16,), seq_len - ignored_len, jnp.int32)

    x_flat = x.reshape(R * d)
    emb_flat = emb_table.reshape(3 * d)
    out = _make_sc_call(R, n, d)(x_flat, emb_flat, us16, bs16, is16)
    return out.reshape(batch, n, d)


# speedup vs baseline: 1.7272x; 1.3184x over previous
import jax
import jax.numpy as jnp
from jax import lax
from jax.experimental import pallas as pl
from jax.experimental.pallas import tpu as pltpu
from jax.experimental.pallas import tpu_sc as plsc

_NC = 2
_NS = 16
_NW = _NC * _NS
_CB = 32
_SB = 4


def _compute_chunk(buf, tab, uv, bv, iv, cpos0, d):
    def sub_block(s, carry):
        base = s * (_SB * d)
        masks = []
        for r in range(_SB):
            p16 = jnp.full((16,), cpos0 + s * _SB + r, jnp.int32)
            masks.append((p16 < uv, p16 < bv, p16 < iv))

        @plsc.parallel_loop(0, d // 16, unroll=2)
        def _cols(k):
            off = k * 16
            t0k = tab[pl.ds(off, 16)]
            t1k = tab[pl.ds(d + off, 16)]
            t2k = tab[pl.ds(2 * d + off, 16)]
            for r in range(_SB):
                c1, c2, c3 = masks[r]
                e = jnp.where(c1, t0k, jnp.where(c2, t1k, jnp.where(c3, t0k, t2k)))
                plsc.addupdate(buf.at[pl.ds(base + r * d + off, 16)], e)

        return carry

    lax.fori_loop(0, _CB // _SB, sub_block, 0)


def _make_sc_call(R, n, d):
    rows_pw = R // _NW
    nch = rows_pw // _CB
    mesh = plsc.VectorSubcoreMesh(
        core_axis_name="c", subcore_axis_name="s", num_cores=_NC, num_subcores=_NS
    )

    def body(x_hbm, emb_hbm, us_hbm, bs_hbm, is_hbm, out_hbm,
             xbuf0, xbuf1, tab, usv, bsv, isv,
             semi0, semi1, semo0, semo1):
        wid = lax.axis_index("s") * _NC + lax.axis_index("c")
        row0 = wid * rows_pw
        pos0 = lax.rem(row0, n)

        pltpu.sync_copy(emb_hbm, tab)
        pltpu.sync_copy(us_hbm, usv)
        pltpu.sync_copy(bs_hbm, bsv)
        pltpu.sync_copy(is_hbm, isv)
        uv = usv[...]
        bv = bsv[...]
        iv = isv[...]

        bufs = (xbuf0, xbuf1)
        semis = (semi0, semi1)
        semos = (semo0, semo1)

        def start_in(c, buf, sem):
            start = (row0 + c * _CB) * d
            return pltpu.async_copy(x_hbm.at[pl.ds(start, _CB * d)], buf, sem)

        def start_out(c, buf, sem):
            start = (row0 + c * _CB) * d
            return pltpu.async_copy(buf, out_hbm.at[pl.ds(start, _CB * d)], sem)

        def wait_in(c, buf, sem):
            start = (row0 + c * _CB) * d
            pltpu.make_async_copy(x_hbm.at[pl.ds(start, _CB * d)], buf, sem).wait()

        def wait_out(c, buf, sem):
            start = (row0 + c * _CB) * d
            pltpu.make_async_copy(buf, out_hbm.at[pl.ds(start, _CB * d)], sem).wait()

        start_in(0, bufs[0], semis[0])

        def group(g, carry):
            for b in range(2):
                c = 2 * g + b
                wait_in(c, bufs[b], semis[b])
                nxt = 1 - b
                if b == 0:
                    @pl.when(g >= 1)
                    def _():
                        wait_out(c - 1, bufs[nxt], semos[nxt])
                    start_in(c + 1, bufs[nxt], semis[nxt])
                else:
                    wait_out(c - 1, bufs[nxt], semos[nxt])

                    @pl.when(g < nch // 2 - 1)
                    def _():
                        start_in(c + 1, bufs[nxt], semis[nxt])
                _compute_chunk(bufs[b], tab, uv, bv, iv, pos0 + c * _CB, d)
                start_out(c, bufs[b], semos[b])
            return carry

        lax.fori_loop(0, nch // 2, group, 0)
        wait_out(nch - 1, bufs[1], semos[1])

    return pl.kernel(
        body,
        out_type=jax.ShapeDtypeStruct((R * d,), jnp.float32),
        mesh=mesh,
        scratch_types=[
            pltpu.VMEM((_CB * d,), jnp.float32),
            pltpu.VMEM((_CB * d,), jnp.float32),
            pltpu.VMEM((3 * d,), jnp.float32),
            pltpu.VMEM((16,), jnp.int32),
            pltpu.VMEM((16,), jnp.int32),
            pltpu.VMEM((16,), jnp.int32),
            pltpu.SemaphoreType.DMA,
            pltpu.SemaphoreType.DMA,
            pltpu.SemaphoreType.DMA,
            pltpu.SemaphoreType.DMA,
        ],
    )


def kernel(x, emb_table, seq_len, front, back, keyframe_gap):
    batch, n, d = x.shape
    R = batch * n
    seq_len = jnp.asarray(seq_len, jnp.int32)
    front = jnp.asarray(front, jnp.int32)
    back = jnp.asarray(back, jnp.int32)
    keyframe_gap = jnp.asarray(keyframe_gap, jnp.int32)
    ignored_len = seq_len - front - back - keyframe_gap
    us16 = jnp.full((16,), front, jnp.int32)
    bs16 = jnp.full((16,), front + keyframe_gap, jnp.int32)
    is16 = jnp.full((16,), seq_len - ignored_len, jnp.int32)

    x_flat = x.reshape(R * d)
    emb_flat = emb_table.reshape(3 * d)
    out = _make_sc_call(R, n, d)(x_flat, emb_flat, us16, bs16, is16)
    return out.reshape(batch, n, d)


# speedup vs baseline: 1.7581x; 1.0179x over previous
import jax
import jax.numpy as jnp
from jax import lax
from jax.experimental import pallas as pl
from jax.experimental.pallas import tpu as pltpu
from jax.experimental.pallas import tpu_sc as plsc

_NC = 2
_NS = 16
_NW = _NC * _NS
_CB = 16
_SB = 4


def _compute_chunk(src, dst, tab, uv, bv, iv, cpos0, d):
    def sub_block(s, carry):
        base = s * (_SB * d)
        masks = []
        for r in range(_SB):
            p16 = jnp.full((16,), cpos0 + s * _SB + r, jnp.int32)
            masks.append((p16 < uv, p16 < bv, p16 < iv))

        @plsc.parallel_loop(0, d // 16, unroll=2)
        def _cols(k):
            off = k * 16
            t0k = tab[pl.ds(off, 16)]
            t1k = tab[pl.ds(d + off, 16)]
            t2k = tab[pl.ds(2 * d + off, 16)]
            for r in range(_SB):
                c1, c2, c3 = masks[r]
                e = jnp.where(c1, t0k, jnp.where(c2, t1k, jnp.where(c3, t0k, t2k)))
                o = base + r * d + off
                dst[pl.ds(o, 16)] = src[pl.ds(o, 16)] + e

        return carry

    lax.fori_loop(0, _CB // _SB, sub_block, 0)


def _make_sc_call(R, n, d):
    rows_pw = R // _NW
    nch = rows_pw // _CB
    mesh = plsc.VectorSubcoreMesh(
        core_axis_name="c", subcore_axis_name="s", num_cores=_NC, num_subcores=_NS
    )

    def body(x_hbm, emb_hbm, us_hbm, bs_hbm, is_hbm, out_hbm,
             ibuf0, ibuf1, obuf0, obuf1, tab, usv, bsv, isv,
             semi0, semi1, semo0, semo1):
        wid = lax.axis_index("s") * _NC + lax.axis_index("c")
        row0 = wid * rows_pw
        pos0 = lax.rem(row0, n)

        pltpu.sync_copy(emb_hbm, tab)
        pltpu.sync_copy(us_hbm, usv)
        pltpu.sync_copy(bs_hbm, bsv)
        pltpu.sync_copy(is_hbm, isv)
        uv = usv[...]
        bv = bsv[...]
        iv = isv[...]

        ibufs = (ibuf0, ibuf1)
        obufs = (obuf0, obuf1)
        semis = (semi0, semi1)
        semos = (semo0, semo1)

        def start_in(c, buf, sem):
            start = (row0 + c * _CB) * d
            return pltpu.async_copy(x_hbm.at[pl.ds(start, _CB * d)], buf, sem)

        def start_out(c, buf, sem):
            start = (row0 + c * _CB) * d
            return pltpu.async_copy(buf, out_hbm.at[pl.ds(start, _CB * d)], sem)

        def wait_in(c, buf, sem):
            start = (row0 + c * _CB) * d
            pltpu.make_async_copy(x_hbm.at[pl.ds(start, _CB * d)], buf, sem).wait()

        def wait_out(c, buf, sem):
            start = (row0 + c * _CB) * d
            pltpu.make_async_copy(buf, out_hbm.at[pl.ds(start, _CB * d)], sem).wait()

        start_in(0, ibufs[0], semis[0])
        start_in(1, ibufs[1], semis[1])

        def group(g, carry):
            for b in range(2):
                c = 2 * g + b
                wait_in(c, ibufs[b], semis[b])

                @pl.when(g >= 1)
                def _():
                    wait_out(c - 2, obufs[b], semos[b])

                _compute_chunk(ibufs[b], obufs[b], tab, uv, bv, iv,
                               pos0 + c * _CB, d)
                start_out(c, obufs[b], semos[b])

                @pl.when(g < nch // 2 - 1)
                def _():
                    start_in(c + 2, ibufs[b], semis[b])
            return carry

        lax.fori_loop(0, nch // 2, group, 0)
        wait_out(nch - 2, obufs[0], semos[0])
        wait_out(nch - 1, obufs[1], semos[1])

    return pl.kernel(
        body,
        out_type=jax.ShapeDtypeStruct((R * d,), jnp.float32),
        mesh=mesh,
        scratch_types=[
            pltpu.VMEM((_CB * d,), jnp.float32),
            pltpu.VMEM((_CB * d,), jnp.float32),
            pltpu.VMEM((_CB * d,), jnp.float32),
            pltpu.VMEM((_CB * d,), jnp.float32),
            pltpu.VMEM((3 * d,), jnp.float32),
            pltpu.VMEM((16,), jnp.int32),
            pltpu.VMEM((16,), jnp.int32),
            pltpu.VMEM((16,), jnp.int32),
            pltpu.SemaphoreType.DMA,
            pltpu.SemaphoreType.DMA,
            pltpu.SemaphoreType.DMA,
            pltpu.SemaphoreType.DMA,
        ],
    )


def kernel(x, emb_table, seq_len, front, back, keyframe_gap):
    batch, n, d = x.shape
    R = batch * n
    seq_len = jnp.asarray(seq_len, jnp.int32)
    front = jnp.asarray(front, jnp.int32)
    back = jnp.asarray(back, jnp.int32)
    keyframe_gap = jnp.asarray(keyframe_gap, jnp.int32)
    ignored_len = seq_len - front - back - keyframe_gap
    us16 = jnp.full((16,), front, jnp.int32)
    bs16 = jnp.full((16,), front + keyframe_gap, jnp.int32)
    is16 = jnp.full((16,), seq_len - ignored_len, jnp.int32)

    x_flat = x.reshape(R * d)
    emb_flat = emb_table.reshape(3 * d)
    out = _make_sc_call(R, n, d)(x_flat, emb_flat, us16, bs16, is16)
    return out.reshape(batch, n, d)


# speedup vs baseline: 8.0153x; 4.5590x over previous
import jax
import jax.numpy as jnp
from jax.experimental import pallas as pl
from jax.experimental.pallas import tpu as pltpu

_CHUNK = 512
_NB = 4


def _emb_block(bounds_ref, emb_ref, pos0, rows):
    pos = jax.lax.broadcasted_iota(jnp.int32, (rows, 1), 0) + pos0
    t0 = emb_ref[0:1, :]
    t1 = emb_ref[1:2, :]
    t2 = emb_ref[2:3, :]
    return jnp.where(
        pos < bounds_ref[0],
        t0,
        jnp.where(pos < bounds_ref[1], t1, jnp.where(pos < bounds_ref[2], t0, t2)),
    )


def _make_body(R, n, d):
    nch = R // _CHUNK

    def body(bounds_ref, x_hbm, emb_ref, out_hbm, ibufs, obufs, isems, osems):
        def start_in(c):
            b = c % _NB
            pltpu.make_async_copy(
                x_hbm.at[pl.ds(c * _CHUNK, _CHUNK), :], ibufs.at[b], isems.at[b]
            ).start()

        def wait_in(c):
            b = c % _NB
            pltpu.make_async_copy(
                x_hbm.at[pl.ds(c * _CHUNK, _CHUNK), :], ibufs.at[b], isems.at[b]
            ).wait()

        def start_out(c):
            b = c % _NB
            pltpu.make_async_copy(
                obufs.at[b], out_hbm.at[pl.ds(c * _CHUNK, _CHUNK), :], osems.at[b]
            ).start()

        def wait_out(c):
            b = c % _NB
            pltpu.make_async_copy(
                obufs.at[b], out_hbm.at[pl.ds(c * _CHUNK, _CHUNK), :], osems.at[b]
            ).wait()

        for c in range(min(_NB, nch)):
            start_in(c)
        for c in range(nch):
            b = c % _NB
            wait_in(c)
            if c >= _NB:
                wait_out(c - _NB)
            pos0 = (c * _CHUNK) % n
            obufs[b] = ibufs[b] + _emb_block(bounds_ref, emb_ref, pos0, _CHUNK)
            start_out(c)
            if c + _NB < nch:
                start_in(c + _NB)
        for c in range(max(nch - _NB, 0), nch):
            wait_out(c)

    return body


def kernel(x, emb_table, seq_len, front, back, keyframe_gap):
    batch, n, d = x.shape
    R = batch * n
    seq_len = jnp.asarray(seq_len, jnp.int32)
    front = jnp.asarray(front, jnp.int32)
    back = jnp.asarray(back, jnp.int32)
    keyframe_gap = jnp.asarray(keyframe_gap, jnp.int32)
    ignored_len = seq_len - front - back - keyframe_gap
    bounds = jnp.stack(
        [front, front + keyframe_gap, seq_len - ignored_len], axis=0
    ).astype(jnp.int32)

    out = pl.pallas_call(
        _make_body(R, n, d),
        in_specs=[
            pl.BlockSpec(memory_space=pltpu.SMEM),
            pl.BlockSpec(memory_space=pl.ANY),
            pl.BlockSpec(memory_space=pltpu.VMEM),
        ],
        out_specs=pl.BlockSpec(memory_space=pl.ANY),
        out_shape=jax.ShapeDtypeStruct((R, d), x.dtype),
        scratch_shapes=[
            pltpu.VMEM((_NB, _CHUNK, d), jnp.float32),
            pltpu.VMEM((_NB, _CHUNK, d), jnp.float32),
            pltpu.SemaphoreType.DMA((_NB,)),
            pltpu.SemaphoreType.DMA((_NB,)),
        ],
    )(bounds, x.reshape(R, d), emb_table)
    return out.reshape(batch, n, d)


# speedup vs baseline: 8.0468x; 1.0039x over previous
import jax
import jax.numpy as jnp
from jax.experimental import pallas as pl
from jax.experimental.pallas import tpu as pltpu

_CHUNK = 1024
_NB = 4


def _emb_block(bounds_ref, emb_ref, pos0, rows):
    pos = jax.lax.broadcasted_iota(jnp.int32, (rows, 1), 0) + pos0
    t0 = emb_ref[0:1, :]
    t1 = emb_ref[1:2, :]
    t2 = emb_ref[2:3, :]
    return jnp.where(
        pos < bounds_ref[0],
        t0,
        jnp.where(pos < bounds_ref[1], t1, jnp.where(pos < bounds_ref[2], t0, t2)),
    )


def _make_body(R, n, d):
    nch = R // _CHUNK

    def body(bounds_ref, x_hbm, emb_ref, out_hbm, ibufs, obufs, isems, osems):
        def start_in(c):
            b = c % _NB
            pltpu.make_async_copy(
                x_hbm.at[pl.ds(c * _CHUNK, _CHUNK), :], ibufs.at[b], isems.at[b]
            ).start()

        def wait_in(c):
            b = c % _NB
            pltpu.make_async_copy(
                x_hbm.at[pl.ds(c * _CHUNK, _CHUNK), :], ibufs.at[b], isems.at[b]
            ).wait()

        def start_out(c):
            b = c % _NB
            pltpu.make_async_copy(
                obufs.at[b], out_hbm.at[pl.ds(c * _CHUNK, _CHUNK), :], osems.at[b]
            ).start()

        def wait_out(c):
            b = c % _NB
            pltpu.make_async_copy(
                obufs.at[b], out_hbm.at[pl.ds(c * _CHUNK, _CHUNK), :], osems.at[b]
            ).wait()

        for c in range(min(_NB, nch)):
            start_in(c)
        for c in range(nch):
            b = c % _NB
            wait_in(c)
            if c >= _NB:
                wait_out(c - _NB)
            pos0 = (c * _CHUNK) % n
            obufs[b] = ibufs[b] + _emb_block(bounds_ref, emb_ref, pos0, _CHUNK)
            start_out(c)
            if c + _NB < nch:
                start_in(c + _NB)
        for c in range(max(nch - _NB, 0), nch):
            wait_out(c)

    return body


def kernel(x, emb_table, seq_len, front, back, keyframe_gap):
    batch, n, d = x.shape
    R = batch * n
    seq_len = jnp.asarray(seq_len, jnp.int32)
    front = jnp.asarray(front, jnp.int32)
    back = jnp.asarray(back, jnp.int32)
    keyframe_gap = jnp.asarray(keyframe_gap, jnp.int32)
    ignored_len = seq_len - front - back - keyframe_gap
    bounds = jnp.stack(
        [front, front + keyframe_gap, seq_len - ignored_len], axis=0
    ).astype(jnp.int32)

    out = pl.pallas_call(
        _make_body(R, n, d),
        in_specs=[
            pl.BlockSpec(memory_space=pltpu.SMEM),
            pl.BlockSpec(memory_space=pl.ANY),
            pl.BlockSpec(memory_space=pltpu.VMEM),
        ],
        out_specs=pl.BlockSpec(memory_space=pl.ANY),
        out_shape=jax.ShapeDtypeStruct((R, d), x.dtype),
        scratch_shapes=[
            pltpu.VMEM((_NB, _CHUNK, d), jnp.float32),
            pltpu.VMEM((_NB, _CHUNK, d), jnp.float32),
            pltpu.SemaphoreType.DMA((_NB,)),
            pltpu.SemaphoreType.DMA((_NB,)),
        ],
    )(bounds, x.reshape(R, d), emb_table)
    return out.reshape(batch, n, d)
